# Initial kernel scaffold; baseline (speedup 1.0000x reference)
import jax
import jax.numpy as jnp
from jax.experimental import pallas as pl


def kernel(x, edge_index, W1, b1, W2, b2, Wc, bc, Wf, bf, Wa, ba):
    raise NotImplementedError("write your pallas kernel here")


# speedup vs baseline: 39.1217x; 39.1217x over previous
import functools

import jax
import jax.numpy as jnp
from jax import lax
from jax.experimental import pallas as pl
from jax.experimental.pallas import tpu as pltpu
from jax.experimental.pallas import tpu_sc as plsc

N = 50000
E = 1600000
NS = 16
NC = 2
ROWS_PER_SUB = N // NS
DEG_CHUNK = 2000
SEG_CHUNK = 1000
BN = 2000
GRID_N = N // BN

_sc_mesh = functools.partial(
    plsc.VectorSubcoreMesh, core_axis_name="c", subcore_axis_name="s",
    num_cores=NC, num_subcores=NS)


@functools.cache
def _deg_kernel_built():
    return functools.partial(
        pl.kernel,
        out_type=jax.ShapeDtypeStruct((NC, N, 16), jnp.float32),
        mesh=_sc_mesh(),
        scratch_types=[
            pltpu.VMEM_SHARED((N, 16), jnp.float32),
            pltpu.VMEM((DEG_CHUNK,), jnp.int32),
            pltpu.VMEM((DEG_CHUNK, 16), jnp.float32),
        ],
        compiler_params=pltpu.CompilerParams(use_tc_tiling_on_sc=False),
    )(_deg_body)


def _deg_body(col_hbm, zeros_hbm, ones_hbm, out_hbm, acc_sh, col_v, ones_v):
    c = lax.axis_index("c")
    s = lax.axis_index("s")
    w = c * NS + s
    r0 = s * ROWS_PER_SUB
    pltpu.sync_copy(ones_hbm, ones_v)
    pltpu.sync_copy(zeros_hbm.at[pl.ds(r0, ROWS_PER_SUB)],
                    acc_sh.at[pl.ds(r0, ROWS_PER_SUB)])
    plsc.subcore_barrier()

    edges_per_worker = E // (NC * NS)

    def body(i, carry):
        base = w * edges_per_worker + i * DEG_CHUNK
        pltpu.sync_copy(col_hbm.at[pl.ds(base, DEG_CHUNK)], col_v)
        pltpu.sync_copy(ones_v, acc_sh.at[col_v], add=True)
        return carry

    lax.fori_loop(0, edges_per_worker // DEG_CHUNK, body, 0)
    plsc.subcore_barrier()
    pltpu.sync_copy(acc_sh.at[pl.ds(r0, ROWS_PER_SUB)],
                    out_hbm.at[c, pl.ds(r0, ROWS_PER_SUB)])


@functools.cache
def _seg_kernel_built():
    return functools.partial(
        pl.kernel,
        out_type=jax.ShapeDtypeStruct((NC, N, 16), jnp.float32),
        mesh=_sc_mesh(),
        scratch_types=[
            pltpu.VMEM_SHARED((N, 16), jnp.float32),
            pltpu.VMEM_SHARED((N, 16), jnp.float32),
            pltpu.VMEM((SEG_CHUNK,), jnp.int32),
            pltpu.VMEM((SEG_CHUNK,), jnp.int32),
            pltpu.VMEM((SEG_CHUNK, 16), jnp.float32),
            pltpu.SemaphoreType.DMA,
        ],
        compiler_params=pltpu.CompilerParams(use_tc_tiling_on_sc=False),
    )(_seg_body)


def _seg_body(y_hbm, row_hbm, col_hbm, zeros_hbm, out_hbm,
              table_sh, acc_sh, row_v, col_v, upd_v, sem):
    c = lax.axis_index("c")
    s = lax.axis_index("s")
    r0 = s * ROWS_PER_SUB
    pltpu.sync_copy(y_hbm.at[c, pl.ds(r0, ROWS_PER_SUB)],
                    table_sh.at[pl.ds(r0, ROWS_PER_SUB)])
    pltpu.sync_copy(zeros_hbm.at[pl.ds(r0, ROWS_PER_SUB)],
                    acc_sh.at[pl.ds(r0, ROWS_PER_SUB)])
    plsc.subcore_barrier()

    edges_per_sub = E // NS

    def body(i, carry):
        base = s * edges_per_sub + i * SEG_CHUNK
        pltpu.sync_copy(row_hbm.at[pl.ds(base, SEG_CHUNK)], row_v)
        pltpu.sync_copy(col_hbm.at[pl.ds(base, SEG_CHUNK)], col_v)
        pltpu.async_copy(table_sh.at[row_v], upd_v, sem).wait()
        pltpu.sync_copy(upd_v, acc_sh.at[col_v], add=True)
        return carry

    lax.fori_loop(0, edges_per_sub // SEG_CHUNK, body, 0)
    plsc.subcore_barrier()
    pltpu.sync_copy(acc_sh.at[pl.ds(r0, ROWS_PER_SUB)],
                    out_hbm.at[c, pl.ds(r0, ROWS_PER_SUB)])


def _t1_body(degp_ref, x_ref, w1_ref, dis_ref, y_ref):
    deg = degp_ref[0][:, 0:1] + degp_ref[1][:, 0:1]
    dis = jnp.where(deg > 0.0, lax.rsqrt(deg), 0.0)
    x = x_ref[...]
    w1 = w1_ref[...]
    xw = (x[:, 0:1] * w1[0:1, :]
          + x[:, 1:2] * w1[1:2, :]
          + x[:, 2:3] * w1[2:3, :])
    y = xw * dis
    dis_ref[...] = dis
    y_ref[0] = y[:, :16]
    y_ref[1] = y[:, 16:]


def _t1(degp, x, W1):
    return pl.pallas_call(
        _t1_body,
        grid=(GRID_N,),
        in_specs=[
            pl.BlockSpec((NC, BN, 16), lambda i: (0, i, 0)),
            pl.BlockSpec((BN, 3), lambda i: (i, 0)),
            pl.BlockSpec((3, 32), lambda i: (0, 0)),
        ],
        out_specs=[
            pl.BlockSpec((BN, 1), lambda i: (i, 0)),
            pl.BlockSpec((NC, BN, 16), lambda i: (0, i, 0)),
        ],
        out_shape=[
            jax.ShapeDtypeStruct((N, 1), jnp.float32),
            jax.ShapeDtypeStruct((NC, N, 16), jnp.float32),
        ],
    )(degp, x, W1)


def _t2_body(s_ref, dis_ref, b1_ref, w2_ref, y_ref):
    sfull = jnp.concatenate([s_ref[0], s_ref[1]], axis=1)
    dis = dis_ref[...]
    h = jnp.maximum(sfull * dis + b1_ref[...], 0.0)
    y = jnp.dot(h, w2_ref[...], preferred_element_type=jnp.float32) * dis
    y_ref[0] = y[:, :16]
    y_ref[1] = y[:, 16:]


def _t2(s1, dis, b1, W2):
    return pl.pallas_call(
        _t2_body,
        grid=(GRID_N,),
        in_specs=[
            pl.BlockSpec((NC, BN, 16), lambda i: (0, i, 0)),
            pl.BlockSpec((BN, 1), lambda i: (i, 0)),
            pl.BlockSpec((1, 32), lambda i: (0, 0)),
            pl.BlockSpec((32, 32), lambda i: (0, 0)),
        ],
        out_specs=[pl.BlockSpec((NC, BN, 16), lambda i: (0, i, 0))],
        out_shape=[jax.ShapeDtypeStruct((NC, N, 16), jnp.float32)],
    )(s1, dis, b1, W2)[0]


def _t3_body(s_ref, dis_ref, b2_ref, wc_ref, bc_ref, wf_ref, bf_ref,
             wa_ref, ba_ref, chase_ref, flee_ref, act_ref):
    i = pl.program_id(0)
    sfull = jnp.concatenate([s_ref[0], s_ref[1]], axis=1)
    dis = dis_ref[...]
    h = jnp.maximum(sfull * dis + b2_ref[...], 0.0)
    chase_ref[...] = (
        jnp.dot(h, wc_ref[...], preferred_element_type=jnp.float32)
        + bc_ref[...])
    flee_ref[...] = (
        jnp.dot(h, wf_ref[...], preferred_element_type=jnp.float32)
        + bf_ref[...])

    @pl.when(i == GRID_N - 1)
    def _():
        act_ref[...] = (
            jnp.dot(h[BN - 1:BN, :], wa_ref[...],
                    preferred_element_type=jnp.float32) + ba_ref[...])


def _t3(s2, dis, b2, Wc, bc, Wf, bf, Wa, ba):
    return pl.pallas_call(
        _t3_body,
        grid=(GRID_N,),
        in_specs=[
            pl.BlockSpec((NC, BN, 16), lambda i: (0, i, 0)),
            pl.BlockSpec((BN, 1), lambda i: (i, 0)),
            pl.BlockSpec((1, 32), lambda i: (0, 0)),
            pl.BlockSpec((32, 1), lambda i: (0, 0)),
            pl.BlockSpec((1, 1), lambda i: (0, 0)),
            pl.BlockSpec((32, 1), lambda i: (0, 0)),
            pl.BlockSpec((1, 1), lambda i: (0, 0)),
            pl.BlockSpec((32, 8), lambda i: (0, 0)),
            pl.BlockSpec((1, 8), lambda i: (0, 0)),
        ],
        out_specs=[
            pl.BlockSpec((BN, 1), lambda i: (i, 0)),
            pl.BlockSpec((BN, 1), lambda i: (i, 0)),
            pl.BlockSpec((1, 8), lambda i: (0, 0)),
        ],
        out_shape=[
            jax.ShapeDtypeStruct((N, 1), jnp.float32),
            jax.ShapeDtypeStruct((N, 1), jnp.float32),
            jax.ShapeDtypeStruct((1, 8), jnp.float32),
        ],
    )(s2, dis, b2, Wc, bc, Wf, bf, Wa, ba)


def kernel(x, edge_index, W1, b1, W2, b2, Wc, bc, Wf, bf, Wa, ba):
    row = edge_index[0]
    col = edge_index[1]
    zeros_n16 = jnp.zeros((N, 16), jnp.float32)
    ones_chunk = jnp.ones((DEG_CHUNK, 16), jnp.float32)

    degp = _deg_kernel_built()(col, zeros_n16, ones_chunk)
    dis, y1 = _t1(degp, x, W1)
    s1 = _seg_kernel_built()(y1, row, col, zeros_n16)
    y2 = _t2(s1, dis, b1.reshape(1, 32), W2)
    s2 = _seg_kernel_built()(y2, row, col, zeros_n16)
    chase, flee, act = _t3(s2, dis, b2.reshape(1, 32),
                           Wc, bc.reshape(1, 1), Wf, bf.reshape(1, 1),
                           Wa, ba.reshape(1, 8))
    return chase[:, 0], flee[:, 0], act[0]


# speedup vs baseline: 47.0882x; 1.2036x over previous
import functools

import jax
import jax.numpy as jnp
from jax import lax
from jax.experimental import pallas as pl
from jax.experimental.pallas import tpu as pltpu
from jax.experimental.pallas import tpu_sc as plsc

N = 50000
E = 1600000
NS = 16
NC = 2
ROWS_PER_SUB = N // NS
DEG_CHUNK = 1000
SEG_CHUNK = 2000
BN = 2000
GRID_N = N // BN

_sc_mesh = functools.partial(
    plsc.VectorSubcoreMesh, core_axis_name="c", subcore_axis_name="s",
    num_cores=NC, num_subcores=NS)


@functools.cache
def _deg_kernel_built():
    return functools.partial(
        pl.kernel,
        out_type=jax.ShapeDtypeStruct((NC, N, 16), jnp.float32),
        mesh=_sc_mesh(),
        scratch_types=[
            pltpu.VMEM_SHARED((N, 16), jnp.float32),
            pltpu.VMEM((DEG_CHUNK,), jnp.int32),
            pltpu.VMEM((DEG_CHUNK,), jnp.int32),
            pltpu.VMEM((DEG_CHUNK, 16), jnp.float32),
            pltpu.SemaphoreType.DMA,
            pltpu.SemaphoreType.DMA,
            pltpu.SemaphoreType.DMA,
            pltpu.SemaphoreType.DMA,
        ],
        compiler_params=pltpu.CompilerParams(use_tc_tiling_on_sc=False),
    )(_deg_body)


def _deg_body(col_hbm, zeros_hbm, ones_hbm, out_hbm, acc_sh,
              col_a, col_b, ones_v, cs_a, cs_b, ss_a, ss_b):
    c = lax.axis_index("c")
    s = lax.axis_index("s")
    w = c * NS + s
    r0 = s * ROWS_PER_SUB
    pltpu.sync_copy(ones_hbm, ones_v)
    pltpu.sync_copy(zeros_hbm.at[pl.ds(r0, ROWS_PER_SUB)],
                    acc_sh.at[pl.ds(r0, ROWS_PER_SUB)])
    plsc.subcore_barrier()

    edges_per_worker = E // (NC * NS)
    n_pairs = edges_per_worker // (2 * DEG_CHUNK)
    base0 = w * edges_per_worker

    def issue_idx(chunk_a_base):
        pltpu.async_copy(col_hbm.at[pl.ds(chunk_a_base, DEG_CHUNK)], col_a, cs_a)
        pltpu.async_copy(
            col_hbm.at[pl.ds(chunk_a_base + DEG_CHUNK, DEG_CHUNK)], col_b, cs_b)

    issue_idx(base0)

    def body(j, carry):
        base = base0 + j * 2 * DEG_CHUNK
        pltpu.make_async_copy(col_hbm.at[pl.ds(0, DEG_CHUNK)], col_a, cs_a).wait()
        sa = pltpu.async_copy(ones_v, acc_sh.at[col_a], ss_a, add=True)
        pltpu.make_async_copy(col_hbm.at[pl.ds(0, DEG_CHUNK)], col_b, cs_b).wait()
        sb = pltpu.async_copy(ones_v, acc_sh.at[col_b], ss_b, add=True)
        sa.wait()
        sb.wait()

        @pl.when(j < n_pairs - 1)
        def _():
            issue_idx(base + 2 * DEG_CHUNK)

        return carry

    lax.fori_loop(0, n_pairs, body, 0)
    plsc.subcore_barrier()
    pltpu.sync_copy(acc_sh.at[pl.ds(r0, ROWS_PER_SUB)],
                    out_hbm.at[c, pl.ds(r0, ROWS_PER_SUB)])


@functools.cache
def _seg_kernel_built():
    return functools.partial(
        pl.kernel,
        out_type=jax.ShapeDtypeStruct((NC, N, 16), jnp.float32),
        mesh=_sc_mesh(),
        scratch_types=[
            pltpu.VMEM_SHARED((N, 16), jnp.float32),
            pltpu.VMEM((SEG_CHUNK,), jnp.int32),
            pltpu.VMEM((SEG_CHUNK,), jnp.int32),
            pltpu.VMEM((SEG_CHUNK,), jnp.int32),
            pltpu.VMEM((SEG_CHUNK,), jnp.int32),
            pltpu.VMEM((SEG_CHUNK, 16), jnp.float32),
            pltpu.VMEM((SEG_CHUNK, 16), jnp.float32),
            pltpu.SemaphoreType.DMA,
            pltpu.SemaphoreType.DMA,
            pltpu.SemaphoreType.DMA,
            pltpu.SemaphoreType.DMA,
            pltpu.SemaphoreType.DMA,
            pltpu.SemaphoreType.DMA,
            pltpu.SemaphoreType.DMA,
            pltpu.SemaphoreType.DMA,
        ],
        compiler_params=pltpu.CompilerParams(use_tc_tiling_on_sc=False),
    )(_seg_body)


def _seg_body(y_hbm, row_hbm, col_hbm, zeros_hbm, out_hbm, acc_sh,
              row_a, row_b, col_a, col_b, upd_a, upd_b,
              rs_a, rs_b, cs_a, cs_b, gs_a, gs_b, ss_a, ss_b):
    c = lax.axis_index("c")
    s = lax.axis_index("s")
    r0 = s * ROWS_PER_SUB
    pltpu.sync_copy(zeros_hbm.at[pl.ds(r0, ROWS_PER_SUB)],
                    acc_sh.at[pl.ds(r0, ROWS_PER_SUB)])
    plsc.subcore_barrier()

    edges_per_sub = E // NS
    n_pairs = edges_per_sub // (2 * SEG_CHUNK)
    base0 = s * edges_per_sub

    def issue_idx(chunk_a_base):
        pltpu.async_copy(row_hbm.at[pl.ds(chunk_a_base, SEG_CHUNK)], row_a, rs_a)
        pltpu.async_copy(col_hbm.at[pl.ds(chunk_a_base, SEG_CHUNK)], col_a, cs_a)
        pltpu.async_copy(
            row_hbm.at[pl.ds(chunk_a_base + SEG_CHUNK, SEG_CHUNK)], row_b, rs_b)
        pltpu.async_copy(
            col_hbm.at[pl.ds(chunk_a_base + SEG_CHUNK, SEG_CHUNK)], col_b, cs_b)

    issue_idx(base0)

    def body(j, carry):
        base = base0 + j * 2 * SEG_CHUNK
        pltpu.make_async_copy(row_hbm.at[pl.ds(0, SEG_CHUNK)], row_a, rs_a).wait()
        pltpu.make_async_copy(col_hbm.at[pl.ds(0, SEG_CHUNK)], col_a, cs_a).wait()
        ga = pltpu.async_copy(y_hbm.at[c].at[row_a], upd_a, gs_a)
        pltpu.make_async_copy(row_hbm.at[pl.ds(0, SEG_CHUNK)], row_b, rs_b).wait()
        pltpu.make_async_copy(col_hbm.at[pl.ds(0, SEG_CHUNK)], col_b, cs_b).wait()
        gb = pltpu.async_copy(y_hbm.at[c].at[row_b], upd_b, gs_b)
        ga.wait()
        sa = pltpu.async_copy(upd_a, acc_sh.at[col_a], ss_a, add=True)
        gb.wait()
        sb = pltpu.async_copy(upd_b, acc_sh.at[col_b], ss_b, add=True)
        sa.wait()
        sb.wait()

        @pl.when(j < n_pairs - 1)
        def _():
            issue_idx(base + 2 * SEG_CHUNK)

        return carry

    lax.fori_loop(0, n_pairs, body, 0)
    plsc.subcore_barrier()
    pltpu.sync_copy(acc_sh.at[pl.ds(r0, ROWS_PER_SUB)],
                    out_hbm.at[c, pl.ds(r0, ROWS_PER_SUB)])


def _t1_body(degp_ref, x_ref, w1_ref, dis_ref, y_ref):
    deg = degp_ref[0][:, 0:1] + degp_ref[1][:, 0:1]
    dis = jnp.where(deg > 0.0, lax.rsqrt(deg), 0.0)
    x = x_ref[...]
    w1 = w1_ref[...]
    xw = (x[:, 0:1] * w1[0:1, :]
          + x[:, 1:2] * w1[1:2, :]
          + x[:, 2:3] * w1[2:3, :])
    y = xw * dis
    dis_ref[...] = dis
    y_ref[0] = y[:, :16]
    y_ref[1] = y[:, 16:]


def _t1(degp, x, W1):
    return pl.pallas_call(
        _t1_body,
        grid=(GRID_N,),
        in_specs=[
            pl.BlockSpec((NC, BN, 16), lambda i: (0, i, 0)),
            pl.BlockSpec((BN, 3), lambda i: (i, 0)),
            pl.BlockSpec((3, 32), lambda i: (0, 0)),
        ],
        out_specs=[
            pl.BlockSpec((BN, 1), lambda i: (i, 0)),
            pl.BlockSpec((NC, BN, 16), lambda i: (0, i, 0)),
        ],
        out_shape=[
            jax.ShapeDtypeStruct((N, 1), jnp.float32),
            jax.ShapeDtypeStruct((NC, N, 16), jnp.float32),
        ],
    )(degp, x, W1)


def _t2_body(s_ref, dis_ref, b1_ref, w2_ref, y_ref):
    sfull = jnp.concatenate([s_ref[0], s_ref[1]], axis=1)
    dis = dis_ref[...]
    h = jnp.maximum(sfull * dis + b1_ref[...], 0.0)
    y = jnp.dot(h, w2_ref[...], preferred_element_type=jnp.float32,
                precision=lax.Precision.HIGHEST) * dis
    y_ref[0] = y[:, :16]
    y_ref[1] = y[:, 16:]


def _t2(s1, dis, b1, W2):
    return pl.pallas_call(
        _t2_body,
        grid=(GRID_N,),
        in_specs=[
            pl.BlockSpec((NC, BN, 16), lambda i: (0, i, 0)),
            pl.BlockSpec((BN, 1), lambda i: (i, 0)),
            pl.BlockSpec((1, 32), lambda i: (0, 0)),
            pl.BlockSpec((32, 32), lambda i: (0, 0)),
        ],
        out_specs=[pl.BlockSpec((NC, BN, 16), lambda i: (0, i, 0))],
        out_shape=[jax.ShapeDtypeStruct((NC, N, 16), jnp.float32)],
    )(s1, dis, b1, W2)[0]


def _t3_body(s_ref, dis_ref, b2_ref, wc_ref, bc_ref, wf_ref, bf_ref,
             wa_ref, ba_ref, chase_ref, flee_ref, act_ref):
    i = pl.program_id(0)
    sfull = jnp.concatenate([s_ref[0], s_ref[1]], axis=1)
    dis = dis_ref[...]
    h = jnp.maximum(sfull * dis + b2_ref[...], 0.0)
    chase_ref[...] = (
        jnp.dot(h, wc_ref[...], preferred_element_type=jnp.float32,
                precision=lax.Precision.HIGHEST)
        + bc_ref[...])
    flee_ref[...] = (
        jnp.dot(h, wf_ref[...], preferred_element_type=jnp.float32,
                precision=lax.Precision.HIGHEST)
        + bf_ref[...])

    @pl.when(i == GRID_N - 1)
    def _():
        act_ref[...] = (
            jnp.dot(h[BN - 1:BN, :], wa_ref[...],
                    preferred_element_type=jnp.float32,
                precision=lax.Precision.HIGHEST) + ba_ref[...])


def _t3(s2, dis, b2, Wc, bc, Wf, bf, Wa, ba):
    return pl.pallas_call(
        _t3_body,
        grid=(GRID_N,),
        in_specs=[
            pl.BlockSpec((NC, BN, 16), lambda i: (0, i, 0)),
            pl.BlockSpec((BN, 1), lambda i: (i, 0)),
            pl.BlockSpec((1, 32), lambda i: (0, 0)),
            pl.BlockSpec((32, 1), lambda i: (0, 0)),
            pl.BlockSpec((1, 1), lambda i: (0, 0)),
            pl.BlockSpec((32, 1), lambda i: (0, 0)),
            pl.BlockSpec((1, 1), lambda i: (0, 0)),
            pl.BlockSpec((32, 8), lambda i: (0, 0)),
            pl.BlockSpec((1, 8), lambda i: (0, 0)),
        ],
        out_specs=[
            pl.BlockSpec((BN, 1), lambda i: (i, 0)),
            pl.BlockSpec((BN, 1), lambda i: (i, 0)),
            pl.BlockSpec((1, 8), lambda i: (0, 0)),
        ],
        out_shape=[
            jax.ShapeDtypeStruct((N, 1), jnp.float32),
            jax.ShapeDtypeStruct((N, 1), jnp.float32),
            jax.ShapeDtypeStruct((1, 8), jnp.float32),
        ],
    )(s2, dis, b2, Wc, bc, Wf, bf, Wa, ba)


def kernel(x, edge_index, W1, b1, W2, b2, Wc, bc, Wf, bf, Wa, ba):
    row = edge_index[0]
    col = edge_index[1]
    zeros_n16 = jnp.zeros((N, 16), jnp.float32)
    ones_chunk = jnp.ones((DEG_CHUNK, 16), jnp.float32)

    degp = _deg_kernel_built()(col, zeros_n16, ones_chunk)
    dis, y1 = _t1(degp, x, W1)
    s1 = _seg_kernel_built()(y1, row, col, zeros_n16)
    y2 = _t2(s1, dis, b1.reshape(1, 32), W2)
    s2 = _seg_kernel_built()(y2, row, col, zeros_n16)
    chase, flee, act = _t3(s2, dis, b2.reshape(1, 32),
                           Wc, bc.reshape(1, 1), Wf, bf.reshape(1, 1),
                           Wa, ba.reshape(1, 8))
    return chase[:, 0], flee[:, 0], act[0]
